# Initial kernel scaffold; baseline (speedup 1.0000x reference)
import jax
import jax.numpy as jnp
from jax.experimental import pallas as pl


def kernel(x, Wg, W1, W2):
    raise NotImplementedError("write your pallas kernel here")


# speedup vs baseline: 1.3083x; 1.3083x over previous
import functools

import jax
import jax.numpy as jnp
from jax.experimental import pallas as pl
from jax.experimental.pallas import tpu as pltpu

HIDDEN = 768
FF = 3072
E = 8
TOPK = 2

FF_BLK = 512


def _gate_body(h_ref, wg_ref, w_ref):
    h = h_ref[...]
    logits = jnp.dot(h, wg_ref[...], preferred_element_type=jnp.float32)
    ids = jax.lax.broadcasted_iota(jnp.int32, logits.shape, 1)
    m0 = jnp.max(logits, axis=-1, keepdims=True)
    a0 = jnp.min(jnp.where(logits == m0, ids, E), axis=-1, keepdims=True)
    l2 = jnp.where(ids == a0, -jnp.inf, logits)
    m1 = jnp.max(l2, axis=-1, keepdims=True)
    a1 = jnp.min(jnp.where(l2 == m1, ids, E), axis=-1, keepdims=True)
    p1 = jnp.exp(m1 - m0)
    denom = 1.0 + p1
    w0 = 1.0 / denom
    w1 = p1 / denom
    w_ref[...] = jnp.where(ids == a0, w0, 0.0) + jnp.where(ids == a1, w1, 0.0)


def _dense_body(h_ref, w1_ref, w2_ref, wfull_ref, y_ref):
    e = pl.program_id(0)
    f = pl.program_id(1)

    @pl.when(jnp.logical_and(e == 0, f == 0))
    def _():
        y_ref[...] = jnp.zeros_like(y_ref)

    h = h_ref[...].astype(jnp.bfloat16)
    w1 = w1_ref[0].astype(jnp.bfloat16)
    w2 = w2_ref[0].astype(jnp.bfloat16)
    pre = jnp.dot(h, w1, preferred_element_type=jnp.float32)
    act = (pre * jax.nn.sigmoid(pre)).astype(jnp.bfloat16)
    contrib = jnp.dot(act, w2, preferred_element_type=jnp.float32)
    wf = wfull_ref[...]
    lane = jax.lax.broadcasted_iota(jnp.int32, wf.shape, 1)
    wt = jnp.sum(jnp.where(lane == e, wf, 0.0), axis=1, keepdims=True)
    y_ref[...] += wt * contrib


@functools.partial(jax.jit, static_argnames=())
def kernel(x, Wg, W1, W2):
    b, t, d = x.shape
    h = x.reshape(t, d)

    wfull = pl.pallas_call(
        _gate_body,
        out_shape=jax.ShapeDtypeStruct((t, E), jnp.float32),
    )(h, Wg)

    y = pl.pallas_call(
        _dense_body,
        grid=(E, FF // FF_BLK),
        in_specs=[
            pl.BlockSpec((t, d), lambda e, f: (0, 0)),
            pl.BlockSpec((1, d, FF_BLK), lambda e, f: (e, 0, f)),
            pl.BlockSpec((1, FF_BLK, d), lambda e, f: (e, f, 0)),
            pl.BlockSpec((t, E), lambda e, f: (0, 0)),
        ],
        out_specs=pl.BlockSpec((t, d), lambda e, f: (0, 0)),
        out_shape=jax.ShapeDtypeStruct((t, d), jnp.float32),
    )(h, W1, W2, wfull)

    return y.reshape(b, t, d)
